# baseline (speedup 1.0000x reference)
import jax
import jax.numpy as jnp
from jax import lax
from jax.experimental import pallas as pl
from jax.experimental.pallas import tpu as pltpu
from jax.experimental.pallas import tpu_sc as plsc

_N = 50000
_E = 800000
_H = 128
_G = 64
_R = 6400
_EP = _R * 128
_RPT = _R // 16
_R1 = 6656
_EP1 = _R1 * 128
_RPT1 = _R1 // 32
_ACC = 50048
_APT = _ACC // 16
_KB = 16
_RB = 4
_BN = 1000
_NB = _N // _BN

def _sc_mesh():
    return plsc.VectorSubcoreMesh(
        core_axis_name="c", subcore_axis_name="s", num_cores=2, num_subcores=16)


def _edge_pipeline(tbl_hbm, gidx_hbm, didx_hbm, gbase, dbase, ngrp,
                   gidx_v, didx_v, rows_v, acc, sg, ss, semi):
    pltpu.sync_copy(gidx_hbm.at[pl.ds(gbase, _KB)], gidx_v.at[0])
    pltpu.sync_copy(didx_hbm.at[pl.ds(dbase, _KB)], didx_v.at[0])
    plsc.subcore_barrier()

    @pl.loop(0, ngrp)
    def _grp(g):
        b = lax.rem(g, 2)
        nb = 1 - b
        gn = lax.rem(g + 1, ngrp)
        gi = pltpu.async_copy(
            gidx_hbm.at[pl.ds(gbase + gn * _KB, _KB)], gidx_v.at[nb], semi)
        di = pltpu.async_copy(
            didx_hbm.at[pl.ds(dbase + gn * _KB, _KB)], didx_v.at[nb], semi)
        dg = {0: pltpu.async_copy(
            tbl_hbm.at[gidx_v.at[b, 0]], rows_v.at[0], sg[0])}
        for j in range(_KB):
            dg[j].wait()
            if j + 1 < _KB:
                dg[j + 1] = pltpu.async_copy(
                    tbl_hbm.at[gidx_v.at[b, j + 1]],
                    rows_v.at[(j + 1) % 2], sg[(j + 1) % 2])
            pltpu.sync_copy(rows_v.at[j % 2], acc.at[didx_v.at[b, j]],
                            add=True)
        gi.wait()
        di.wait()

    plsc.subcore_barrier()


def _agg16_body(x_hbm, sidx_hbm, didx_hbm, z_hbm, out_hbm,
                sidx_v, didx_v, rows_v, acc,
                sg0, sg1, sg2, sg3, ss0, ss1, ss2, ss3, semi):
    c = lax.axis_index("c")
    s = lax.axis_index("s")
    pltpu.sync_copy(z_hbm, acc.at[pl.ds(s * _APT, _APT)])
    base = c * (16 * _RPT1) + s * _RPT1
    _edge_pipeline(x_hbm, sidx_hbm, didx_hbm, base, base, _RPT1 // _KB,
                   sidx_v, didx_v, rows_v, acc,
                   (sg0, sg1, sg2, sg3), (ss0, ss1, ss2, ss3), semi)
    pltpu.sync_copy(acc.at[pl.ds(s * _APT, _APT)],
                    out_hbm.at[pl.ds(s * _APT, _APT), pl.ds(c * 16, 16)])


def _agg16(x_pad, sidx, didx, z16):
    return pl.kernel(
        _agg16_body,
        out_type=jax.ShapeDtypeStruct((_ACC, 32), jnp.float32),
        mesh=_sc_mesh(),
        scratch_types=[
            pltpu.VMEM((2, _KB, 128), jnp.int32),
            pltpu.VMEM((2, _KB, 128), jnp.int32),
            pltpu.VMEM((_RB, 128, 16), jnp.float32),
            pltpu.VMEM_SHARED((_ACC, 16), jnp.float32),
        ] + [pltpu.SemaphoreType.DMA] * 9,
        compiler_params=pltpu.CompilerParams(use_tc_tiling_on_sc=False),
    )(x_pad, sidx, didx, z16)


def _agg32_body(h_hbm, gidx_hbm, didx_hbm, z_hbm, out_hbm,
                gidx_v, didx_v, rows_v, acc,
                sg0, sg1, sg2, sg3, ss0, ss1, ss2, ss3, semi):
    c = lax.axis_index("c")
    s = lax.axis_index("s")
    for k in range(2):
        cc = c * 2 + k
        pltpu.sync_copy(z_hbm, acc.at[pl.ds(s * _APT, _APT)])
        _edge_pipeline(h_hbm, gidx_hbm, didx_hbm,
                       cc * _R + s * _RPT, s * _RPT, _RPT // _KB,
                       gidx_v, didx_v, rows_v, acc,
                       (sg0, sg1, sg2, sg3), (ss0, ss1, ss2, ss3), semi)
        pltpu.sync_copy(acc.at[pl.ds(s * _APT, _APT)],
                        out_hbm.at[pl.ds(s * _APT, _APT), pl.ds(cc * 32, 32)])


def _agg32(h_flat, gidx, didx, z32):
    return pl.kernel(
        _agg32_body,
        out_type=jax.ShapeDtypeStruct((_ACC, _H), jnp.float32),
        mesh=_sc_mesh(),
        scratch_types=[
            pltpu.VMEM((2, _KB, 128), jnp.int32),
            pltpu.VMEM((2, _KB, 128), jnp.int32),
            pltpu.VMEM((_RB, 128, 32), jnp.float32),
            pltpu.VMEM_SHARED((_ACC, 32), jnp.float32),
        ] + [pltpu.SemaphoreType.DMA] * 9,
        compiler_params=pltpu.CompilerParams(use_tc_tiling_on_sc=False),
    )(h_flat, gidx, didx, z32)


def _bn_ab(s_ref, q_ref, g_ref, be_ref):
    mu = s_ref[...] / float(_N)
    var = q_ref[...] / float(_N) - mu * mu
    a = g_ref[...] * lax.rsqrt(var + 1e-5)
    b = be_ref[...] - mu * a
    return jnp.concatenate([a, b], axis=0)


def _mm1_body(x_ref, a0_ref, w_ref, g_ref, be_ref, t_ref, ab_ref, s_ref, q_ref):
    i = pl.program_id(0)

    @pl.when(i == 0)
    def _():
        s_ref[...] = jnp.zeros_like(s_ref)
        q_ref[...] = jnp.zeros_like(q_ref)

    xa = x_ref[...] + a0_ref[:, :16] + a0_ref[:, 16:]
    t = jnp.dot(xa, w_ref[...], preferred_element_type=jnp.float32)
    t_ref[...] = t
    s_ref[...] += jnp.sum(t, axis=0, keepdims=True)
    q_ref[...] += jnp.sum(t * t, axis=0, keepdims=True)

    @pl.when(i == _NB - 1)
    def _():
        ab_ref[...] = _bn_ab(s_ref, q_ref, g_ref, be_ref)


def _mm1(x_pad, a0, w1p, g, be):
    return pl.pallas_call(
        _mm1_body,
        grid=(_NB,),
        in_specs=[
            pl.BlockSpec((_BN, 16), lambda i: (i, 0)),
            pl.BlockSpec((_BN, 32), lambda i: (i, 0)),
            pl.BlockSpec((16, _H), lambda i: (0, 0)),
            pl.BlockSpec((1, _H), lambda i: (0, 0)),
            pl.BlockSpec((1, _H), lambda i: (0, 0)),
        ],
        out_specs=[
            pl.BlockSpec((_BN, _H), lambda i: (i, 0)),
            pl.BlockSpec((2, _H), lambda i: (0, 0)),
        ],
        out_shape=[
            jax.ShapeDtypeStruct((_N, _H), jnp.float32),
            jax.ShapeDtypeStruct((2, _H), jnp.float32),
        ],
        scratch_shapes=[
            pltpu.VMEM((1, _H), jnp.float32),
            pltpu.VMEM((1, _H), jnp.float32),
        ],
    )(x_pad, a0, w1p, g, be)


def _mm2_body(h_ref, ag_ref, w_ref, g_ref, be_ref, t_ref, ab_ref, s_ref, q_ref):
    i = pl.program_id(0)

    @pl.when(i == 0)
    def _():
        s_ref[...] = jnp.zeros_like(s_ref)
        q_ref[...] = jnp.zeros_like(q_ref)

    t = jnp.dot(h_ref[...] + ag_ref[...], w_ref[...],
                preferred_element_type=jnp.float32)
    t_ref[...] = t
    s_ref[...] += jnp.sum(t, axis=0, keepdims=True)
    q_ref[...] += jnp.sum(t * t, axis=0, keepdims=True)

    @pl.when(i == _NB - 1)
    def _():
        ab_ref[...] = _bn_ab(s_ref, q_ref, g_ref, be_ref)


def _mm2(h4, agg4, w, g, be):
    return pl.pallas_call(
        _mm2_body,
        grid=(_NB,),
        in_specs=[
            pl.BlockSpec((_BN, _H), lambda i: (i, 0)),
            pl.BlockSpec((_BN, _H), lambda i: (i, 0)),
            pl.BlockSpec((_H, _H), lambda i: (0, 0)),
            pl.BlockSpec((1, _H), lambda i: (0, 0)),
            pl.BlockSpec((1, _H), lambda i: (0, 0)),
        ],
        out_specs=[
            pl.BlockSpec((_BN, _H), lambda i: (i, 0)),
            pl.BlockSpec((2, _H), lambda i: (0, 0)),
        ],
        out_shape=[
            jax.ShapeDtypeStruct((_N, _H), jnp.float32),
            jax.ShapeDtypeStruct((2, _H), jnp.float32),
        ],
        scratch_shapes=[
            pltpu.VMEM((1, _H), jnp.float32),
            pltpu.VMEM((1, _H), jnp.float32),
        ],
    )(h4, agg4, w, g, be)


def _act_body(t_ref, ab_ref, o_ref):
    o_ref[...] = jnp.maximum(t_ref[...] * ab_ref[0:1, :] + ab_ref[1:2, :], 0.0)


def _act(t, ab):
    return pl.pallas_call(
        _act_body,
        grid=(_NB,),
        in_specs=[
            pl.BlockSpec((_BN, _H), lambda i: (i, 0)),
            pl.BlockSpec((2, _H), lambda i: (0, 0)),
        ],
        out_specs=pl.BlockSpec((_BN, _H), lambda i: (i, 0)),
        out_shape=jax.ShapeDtypeStruct((_N, _H), jnp.float32),
        compiler_params=pltpu.CompilerParams(
            dimension_semantics=("parallel",)),
    )(t, ab)


def _pool_body(t_ref, ab_ref, b_ref, wfc_ref, o_ref, s_ref, c_ref):
    i = pl.program_id(0)

    @pl.when(i == 0)
    def _():
        s_ref[...] = jnp.zeros_like(s_ref)
        c_ref[...] = jnp.zeros_like(c_ref)

    h = jnp.maximum(t_ref[...] * ab_ref[0:1, :] + ab_ref[1:2, :], 0.0)
    ohT = (lax.broadcasted_iota(jnp.int32, (_G, _BN), 0) == b_ref[0]
           ).astype(jnp.float32)
    s_ref[...] += jnp.dot(ohT, h, preferred_element_type=jnp.float32)
    c_ref[...] += jnp.dot(ohT, jnp.ones_like(h),
                          preferred_element_type=jnp.float32)

    @pl.when(i == _NB - 1)
    def _():
        pooled = s_ref[...] / jnp.maximum(c_ref[...], 1.0)
        z = jnp.dot(pooled, wfc_ref[...], preferred_element_type=jnp.float32)
        o_ref[...] = 1.0 / (1.0 + jnp.exp(-z))


def _pool(t3, ab3, batch_r, wfc_p):
    return pl.pallas_call(
        _pool_body,
        grid=(_NB,),
        in_specs=[
            pl.BlockSpec((_BN, _H), lambda i: (i, 0)),
            pl.BlockSpec((2, _H), lambda i: (0, 0)),
            pl.BlockSpec((1, 1, _BN), lambda i: (i, 0, 0)),
            pl.BlockSpec((_H, _H), lambda i: (0, 0)),
        ],
        out_specs=pl.BlockSpec((_G, _H), lambda i: (0, 0)),
        out_shape=jax.ShapeDtypeStruct((_G, _H), jnp.float32),
        scratch_shapes=[
            pltpu.VMEM((_G, _H), jnp.float32),
            pltpu.VMEM((_G, _H), jnp.float32),
        ],
    )(t3, ab3, batch_r, wfc_p)


def kernel(x, edge_index, batch, W1, g1, b1, W2, g2, b2, W3, g3, b3, Wfc):
    src = edge_index[0]
    dst = edge_index[1]
    pad = _EP - _E
    src_p = jnp.concatenate([src, jnp.zeros((pad,), jnp.int32)])
    dst_p = jnp.concatenate([dst, jnp.full((pad,), _N, jnp.int32)])
    pad1 = _EP1 - _E
    sidx = jnp.concatenate([src, jnp.zeros((pad1,), jnp.int32)]).reshape(_R1, 128)
    didx1 = jnp.concatenate([dst, jnp.full((pad1,), _N, jnp.int32)]).reshape(_R1, 128)
    didx = dst_p.reshape(_R, 128)
    gidx = (src_p[None, :] * 4
            + jnp.arange(4, dtype=jnp.int32)[:, None]).reshape(4 * _R, 128)
    z16 = jnp.zeros((_APT, 16), jnp.float32)
    z32 = jnp.zeros((_APT, 32), jnp.float32)

    x_pad = jnp.pad(x, ((0, 0), (0, 12)))
    w1p = jnp.pad(W1, ((0, 12), (0, 0)))
    wfc_p = jnp.pad(Wfc, ((0, 0), (0, _H - 1)))

    a0 = _agg16(x_pad, sidx, didx1, z16)
    t1, ab1 = _mm1(x_pad, a0, w1p, g1[None], b1[None])
    h1 = _act(t1, ab1)
    agg1 = _agg32(h1.reshape(4 * _N, 32), gidx, didx, z32)
    t2, ab2 = _mm2(h1, agg1, W2, g2[None], b2[None])
    h2 = _act(t2, ab2)
    agg2 = _agg32(h2.reshape(4 * _N, 32), gidx, didx, z32)
    t3, ab3 = _mm2(h2, agg2, W3, g3[None], b3[None])
    out = _pool(t3, ab3, batch.reshape(_NB, 1, _BN), wfc_p)
    return out[:, :1]


# speedup vs baseline: 1.0012x; 1.0012x over previous
import jax
import jax.numpy as jnp
from jax import lax
from jax.experimental import pallas as pl
from jax.experimental.pallas import tpu as pltpu
from jax.experimental.pallas import tpu_sc as plsc

_N = 50000
_E = 800000
_H = 128
_G = 64
_R = 6400
_EP = _R * 128
_RPT = _R // 16
_R1 = 6656
_EP1 = _R1 * 128
_RPT1 = _R1 // 32
_ACC = 51200
_APT = _ACC // 16
_KB = 16
_RB = 4
_BN = 1000
_NB = _N // _BN

def _sc_mesh():
    return plsc.VectorSubcoreMesh(
        core_axis_name="c", subcore_axis_name="s", num_cores=2, num_subcores=16)


def _edge_pipeline(tbl_hbm, gidx_hbm, didx_hbm, gbase, dbase, ngrp,
                   gidx_v, didx_v, rows_v, acc, sg, ss, semi):
    pltpu.sync_copy(gidx_hbm.at[pl.ds(gbase, _KB)], gidx_v.at[0])
    pltpu.sync_copy(didx_hbm.at[pl.ds(dbase, _KB)], didx_v.at[0])
    plsc.subcore_barrier()

    @pl.loop(0, ngrp)
    def _grp(g):
        b = lax.rem(g, 2)
        nb = 1 - b
        gn = lax.rem(g + 1, ngrp)
        gi = pltpu.async_copy(
            gidx_hbm.at[pl.ds(gbase + gn * _KB, _KB)], gidx_v.at[nb], semi)
        di = pltpu.async_copy(
            didx_hbm.at[pl.ds(dbase + gn * _KB, _KB)], didx_v.at[nb], semi)
        dg = {0: pltpu.async_copy(
            tbl_hbm.at[gidx_v.at[b, 0]], rows_v.at[0], sg[0])}
        for j in range(_KB):
            dg[j].wait()
            if j + 1 < _KB:
                dg[j + 1] = pltpu.async_copy(
                    tbl_hbm.at[gidx_v.at[b, j + 1]],
                    rows_v.at[(j + 1) % 2], sg[(j + 1) % 2])
            pltpu.sync_copy(rows_v.at[j % 2], acc.at[didx_v.at[b, j]],
                            add=True)
        gi.wait()
        di.wait()

    plsc.subcore_barrier()


def _agg16_body(x_hbm, sidx_hbm, didx_hbm, z_hbm, out_hbm,
                sidx_v, didx_v, rows_v, acc,
                sg0, sg1, sg2, sg3, ss0, ss1, ss2, ss3, semi):
    c = lax.axis_index("c")
    s = lax.axis_index("s")
    pltpu.sync_copy(z_hbm, acc.at[pl.ds(s * _APT, _APT)])
    base = c * (16 * _RPT1) + s * _RPT1
    _edge_pipeline(x_hbm, sidx_hbm, didx_hbm, base, base, _RPT1 // _KB,
                   sidx_v, didx_v, rows_v, acc,
                   (sg0, sg1, sg2, sg3), (ss0, ss1, ss2, ss3), semi)
    pltpu.sync_copy(acc.at[pl.ds(s * _APT, _APT)],
                    out_hbm.at[pl.ds(s * _APT, _APT), pl.ds(c * 16, 16)])


def _agg16(x_pad, sidx, didx, z16):
    return pl.kernel(
        _agg16_body,
        out_type=jax.ShapeDtypeStruct((_ACC, 32), jnp.float32),
        mesh=_sc_mesh(),
        scratch_types=[
            pltpu.VMEM((2, _KB, 128), jnp.int32),
            pltpu.VMEM((2, _KB, 128), jnp.int32),
            pltpu.VMEM((2, 128, 16), jnp.float32),
            pltpu.VMEM_SHARED((_ACC, 16), jnp.float32),
        ] + [pltpu.SemaphoreType.DMA] * 9,
        compiler_params=pltpu.CompilerParams(use_tc_tiling_on_sc=False),
    )(x_pad, sidx, didx, z16)


def _agg32_body(h_hbm, gidx_hbm, didx_hbm, z_hbm, out_hbm,
                gidx_v, didx_v, rows_v, acc,
                sg0, sg1, sg2, sg3, ss0, ss1, ss2, ss3, semi):
    c = lax.axis_index("c")
    s = lax.axis_index("s")
    for k in range(2):
        cc = c * 2 + k
        pltpu.sync_copy(z_hbm, acc.at[pl.ds(s * _APT, _APT)])
        _edge_pipeline(h_hbm, gidx_hbm, didx_hbm,
                       cc * _R + s * _RPT, s * _RPT, _RPT // _KB,
                       gidx_v, didx_v, rows_v, acc,
                       (sg0, sg1, sg2, sg3), (ss0, ss1, ss2, ss3), semi)
        pltpu.sync_copy(acc.at[pl.ds(s * _APT, _APT)],
                        out_hbm.at[pl.ds(s * _APT, _APT), pl.ds(cc * 32, 32)])


def _agg32(h_flat, gidx, didx, z32):
    return pl.kernel(
        _agg32_body,
        out_type=jax.ShapeDtypeStruct((_ACC, _H), jnp.float32),
        mesh=_sc_mesh(),
        scratch_types=[
            pltpu.VMEM((2, _KB, 128), jnp.int32),
            pltpu.VMEM((2, _KB, 128), jnp.int32),
            pltpu.VMEM((2, 128, 32), jnp.float32),
            pltpu.VMEM_SHARED((_ACC, 32), jnp.float32),
        ] + [pltpu.SemaphoreType.DMA] * 9,
        compiler_params=pltpu.CompilerParams(use_tc_tiling_on_sc=False),
    )(h_flat, gidx, didx, z32)


def _bn_ab(s_ref, q_ref, g_ref, be_ref):
    mu = s_ref[...] / float(_N)
    var = q_ref[...] / float(_N) - mu * mu
    a = g_ref[...] * lax.rsqrt(var + 1e-5)
    b = be_ref[...] - mu * a
    return jnp.concatenate([a, b], axis=0)


def _mm1_body(x_ref, a0_ref, w_ref, g_ref, be_ref, t_ref, ab_ref, s_ref, q_ref):
    i = pl.program_id(0)

    @pl.when(i == 0)
    def _():
        s_ref[...] = jnp.zeros_like(s_ref)
        q_ref[...] = jnp.zeros_like(q_ref)

    xa = x_ref[...] + a0_ref[:, :16] + a0_ref[:, 16:]
    t = jnp.dot(xa, w_ref[...], preferred_element_type=jnp.float32)
    t_ref[...] = t
    s_ref[...] += jnp.sum(t, axis=0, keepdims=True)
    q_ref[...] += jnp.sum(t * t, axis=0, keepdims=True)

    @pl.when(i == _NB - 1)
    def _():
        ab_ref[...] = _bn_ab(s_ref, q_ref, g_ref, be_ref)


def _mm1(x_pad, a0, w1p, g, be):
    return pl.pallas_call(
        _mm1_body,
        grid=(_NB,),
        in_specs=[
            pl.BlockSpec((_BN, 16), lambda i: (i, 0)),
            pl.BlockSpec((_BN, 32), lambda i: (i, 0)),
            pl.BlockSpec((16, _H), lambda i: (0, 0)),
            pl.BlockSpec((1, _H), lambda i: (0, 0)),
            pl.BlockSpec((1, _H), lambda i: (0, 0)),
        ],
        out_specs=[
            pl.BlockSpec((_BN, _H), lambda i: (i, 0)),
            pl.BlockSpec((2, _H), lambda i: (0, 0)),
        ],
        out_shape=[
            jax.ShapeDtypeStruct((_N, _H), jnp.float32),
            jax.ShapeDtypeStruct((2, _H), jnp.float32),
        ],
        scratch_shapes=[
            pltpu.VMEM((1, _H), jnp.float32),
            pltpu.VMEM((1, _H), jnp.float32),
        ],
    )(x_pad, a0, w1p, g, be)


def _mm2_body(h_ref, ag_ref, w_ref, g_ref, be_ref, t_ref, ab_ref, s_ref, q_ref):
    i = pl.program_id(0)

    @pl.when(i == 0)
    def _():
        s_ref[...] = jnp.zeros_like(s_ref)
        q_ref[...] = jnp.zeros_like(q_ref)

    t = jnp.dot(h_ref[...] + ag_ref[...], w_ref[...],
                preferred_element_type=jnp.float32)
    t_ref[...] = t
    s_ref[...] += jnp.sum(t, axis=0, keepdims=True)
    q_ref[...] += jnp.sum(t * t, axis=0, keepdims=True)

    @pl.when(i == _NB - 1)
    def _():
        ab_ref[...] = _bn_ab(s_ref, q_ref, g_ref, be_ref)


def _mm2(h4, agg4, w, g, be):
    return pl.pallas_call(
        _mm2_body,
        grid=(_NB,),
        in_specs=[
            pl.BlockSpec((_BN, _H), lambda i: (i, 0)),
            pl.BlockSpec((_BN, _H), lambda i: (i, 0)),
            pl.BlockSpec((_H, _H), lambda i: (0, 0)),
            pl.BlockSpec((1, _H), lambda i: (0, 0)),
            pl.BlockSpec((1, _H), lambda i: (0, 0)),
        ],
        out_specs=[
            pl.BlockSpec((_BN, _H), lambda i: (i, 0)),
            pl.BlockSpec((2, _H), lambda i: (0, 0)),
        ],
        out_shape=[
            jax.ShapeDtypeStruct((_N, _H), jnp.float32),
            jax.ShapeDtypeStruct((2, _H), jnp.float32),
        ],
        scratch_shapes=[
            pltpu.VMEM((1, _H), jnp.float32),
            pltpu.VMEM((1, _H), jnp.float32),
        ],
    )(h4, agg4, w, g, be)


def _act_body(t_ref, ab_ref, o_ref):
    o_ref[...] = jnp.maximum(t_ref[...] * ab_ref[0:1, :] + ab_ref[1:2, :], 0.0)


def _act(t, ab):
    return pl.pallas_call(
        _act_body,
        grid=(_NB,),
        in_specs=[
            pl.BlockSpec((_BN, _H), lambda i: (i, 0)),
            pl.BlockSpec((2, _H), lambda i: (0, 0)),
        ],
        out_specs=pl.BlockSpec((_BN, _H), lambda i: (i, 0)),
        out_shape=jax.ShapeDtypeStruct((_N, _H), jnp.float32),
        compiler_params=pltpu.CompilerParams(
            dimension_semantics=("parallel",)),
    )(t, ab)


def _pool_body(t_ref, ab_ref, b_ref, wfc_ref, o_ref, s_ref, c_ref):
    i = pl.program_id(0)

    @pl.when(i == 0)
    def _():
        s_ref[...] = jnp.zeros_like(s_ref)
        c_ref[...] = jnp.zeros_like(c_ref)

    h = jnp.maximum(t_ref[...] * ab_ref[0:1, :] + ab_ref[1:2, :], 0.0)
    ohT = (lax.broadcasted_iota(jnp.int32, (_G, _BN), 0) == b_ref[0]
           ).astype(jnp.float32)
    s_ref[...] += jnp.dot(ohT, h, preferred_element_type=jnp.float32)
    c_ref[...] += jnp.dot(ohT, jnp.ones_like(h),
                          preferred_element_type=jnp.float32)

    @pl.when(i == _NB - 1)
    def _():
        pooled = s_ref[...] / jnp.maximum(c_ref[...], 1.0)
        z = jnp.dot(pooled, wfc_ref[...], preferred_element_type=jnp.float32)
        o_ref[...] = 1.0 / (1.0 + jnp.exp(-z))


def _pool(t3, ab3, batch_r, wfc_p):
    return pl.pallas_call(
        _pool_body,
        grid=(_NB,),
        in_specs=[
            pl.BlockSpec((_BN, _H), lambda i: (i, 0)),
            pl.BlockSpec((2, _H), lambda i: (0, 0)),
            pl.BlockSpec((1, 1, _BN), lambda i: (i, 0, 0)),
            pl.BlockSpec((_H, _H), lambda i: (0, 0)),
        ],
        out_specs=pl.BlockSpec((_G, _H), lambda i: (0, 0)),
        out_shape=jax.ShapeDtypeStruct((_G, _H), jnp.float32),
        scratch_shapes=[
            pltpu.VMEM((_G, _H), jnp.float32),
            pltpu.VMEM((_G, _H), jnp.float32),
        ],
    )(t3, ab3, batch_r, wfc_p)


def kernel(x, edge_index, batch, W1, g1, b1, W2, g2, b2, W3, g3, b3, Wfc):
    src = edge_index[0]
    dst = edge_index[1]
    pad = _EP - _E
    src_p = jnp.concatenate([src, jnp.zeros((pad,), jnp.int32)])
    dst_p = jnp.concatenate([dst, jnp.full((pad,), _N, jnp.int32)])
    pad1 = _EP1 - _E
    sidx = jnp.concatenate([src, jnp.zeros((pad1,), jnp.int32)]).reshape(_R1, 128)
    didx1 = jnp.concatenate([dst, jnp.full((pad1,), _N, jnp.int32)]).reshape(_R1, 128)
    didx = dst_p.reshape(_R, 128)
    gidx = (src_p[None, :] * 4
            + jnp.arange(4, dtype=jnp.int32)[:, None]).reshape(4 * _R, 128)
    z16 = jnp.zeros((_APT, 16), jnp.float32)
    z32 = jnp.zeros((_APT, 32), jnp.float32)

    x_pad = jnp.pad(x, ((0, 0), (0, 12)))
    w1p = jnp.pad(W1, ((0, 12), (0, 0)))
    wfc_p = jnp.pad(Wfc, ((0, 0), (0, _H - 1)))

    a0 = _agg16(x_pad, sidx, didx1, z16)
    t1, ab1 = _mm1(x_pad, a0, w1p, g1[None], b1[None])
    h1 = _act(t1, ab1)
    agg1 = _agg32(h1.reshape(4 * _N, 32), gidx, didx, z32)
    t2, ab2 = _mm2(h1, agg1, W2, g2[None], b2[None])
    h2 = _act(t2, ab2)
    agg2 = _agg32(h2.reshape(4 * _N, 32), gidx, didx, z32)
    t3, ab3 = _mm2(h2, agg2, W3, g3[None], b3[None])
    out = _pool(t3, ab3, batch.reshape(_NB, 1, _BN), wfc_p)
    return out[:, :1]


# speedup vs baseline: 1.5470x; 1.5450x over previous
import jax
import jax.numpy as jnp
from jax import lax
from jax.experimental import pallas as pl
from jax.experimental.pallas import tpu as pltpu
from jax.experimental.pallas import tpu_sc as plsc

_N = 50000
_E = 800000
_H = 128
_G = 64
_R = 6272
_EP = _R * 128
_RPT = _R // 16
_R1 = 6400
_EP1 = _R1 * 128
_RPT1 = _R1 // 32
_ACC = 51200
_APT = _ACC // 16
_KB = 8
_BN = 1000
_NB = _N // _BN

def _sc_mesh():
    return plsc.VectorSubcoreMesh(
        core_axis_name="c", subcore_axis_name="s", num_cores=2, num_subcores=16)


def _edge_pipeline(tbl_hbm, gidx_hbm, didx_hbm, gbase, dbase, ngrp,
                   gidx_v, didx_v, rows_v, acc, sg, ss, semi):
    pltpu.sync_copy(gidx_hbm.at[pl.ds(gbase, _KB)], gidx_v.at[0])
    pltpu.sync_copy(didx_hbm.at[pl.ds(dbase, _KB)], didx_v.at[0])
    plsc.subcore_barrier()

    @pl.loop(0, ngrp)
    def _grp(g):
        b = lax.rem(g, 2)
        nb = 1 - b
        gn = lax.rem(g + 1, ngrp)
        gi = pltpu.async_copy(
            gidx_hbm.at[pl.ds(gbase + gn * _KB, _KB)], gidx_v.at[nb], semi)
        di = pltpu.async_copy(
            didx_hbm.at[pl.ds(dbase + gn * _KB, _KB)], didx_v.at[nb], semi)
        dg = {0: pltpu.async_copy(
            tbl_hbm.at[gidx_v.at[b, 0]], rows_v.at[0], sg[0])}
        for j in range(_KB):
            dg[j].wait()
            if j + 1 < _KB:
                dg[j + 1] = pltpu.async_copy(
                    tbl_hbm.at[gidx_v.at[b, j + 1]],
                    rows_v.at[(j + 1) % 2], sg[(j + 1) % 2])
            pltpu.sync_copy(rows_v.at[j % 2], acc.at[didx_v.at[b, j]],
                            add=True)
        gi.wait()
        di.wait()

    plsc.subcore_barrier()


def _agg16_body(x_hbm, sidx_hbm, didx_hbm, z_hbm, out_hbm,
                sidx_v, didx_v, rows_v, acc,
                sg0, sg1, sg2, sg3, ss0, ss1, ss2, ss3, semi):
    c = lax.axis_index("c")
    s = lax.axis_index("s")
    pltpu.sync_copy(z_hbm, acc.at[pl.ds(s * _APT, _APT)])
    base = c * (16 * _RPT1) + s * _RPT1
    _edge_pipeline(x_hbm, sidx_hbm, didx_hbm, base, base, _RPT1 // _KB,
                   sidx_v, didx_v, rows_v, acc,
                   (sg0, sg1, sg2, sg3), (ss0, ss1, ss2, ss3), semi)
    pltpu.sync_copy(acc.at[pl.ds(s * _APT, _APT)],
                    out_hbm.at[pl.ds(s * _APT, _APT), pl.ds(c * 16, 16)])


def _agg16(x_pad, sidx, didx, z16):
    return pl.kernel(
        _agg16_body,
        out_type=jax.ShapeDtypeStruct((_ACC, 32), jnp.float32),
        mesh=_sc_mesh(),
        scratch_types=[
            pltpu.VMEM((2, _KB, 128), jnp.int32),
            pltpu.VMEM((2, _KB, 128), jnp.int32),
            pltpu.VMEM((2, 128, 16), jnp.float32),
            pltpu.VMEM_SHARED((_ACC, 16), jnp.float32),
        ] + [pltpu.SemaphoreType.DMA] * 9,
        compiler_params=pltpu.CompilerParams(use_tc_tiling_on_sc=False),
    )(x_pad, sidx, didx, z16)


def _agg32_body(h_hbm, gidx_hbm, didx_hbm, z_hbm, out_hbm,
                gidx_v, didx_v, rows_v, acc,
                sg0, sg1, sg2, sg3, ss0, ss1, ss2, ss3, semi):
    c = lax.axis_index("c")
    s = lax.axis_index("s")
    for k in range(2):
        cc = c * 2 + k
        pltpu.sync_copy(z_hbm, acc.at[pl.ds(s * _APT, _APT)])
        _edge_pipeline(h_hbm, gidx_hbm, didx_hbm,
                       cc * _R + s * _RPT, s * _RPT, _RPT // _KB,
                       gidx_v, didx_v, rows_v, acc,
                       (sg0, sg1, sg2, sg3), (ss0, ss1, ss2, ss3), semi)
        pltpu.sync_copy(acc.at[pl.ds(s * _APT, _APT)],
                        out_hbm.at[pl.ds(s * _APT, _APT), pl.ds(cc * 32, 32)])


def _agg32(h_flat, gidx, didx, z32):
    return pl.kernel(
        _agg32_body,
        out_type=jax.ShapeDtypeStruct((_ACC, _H), jnp.float32),
        mesh=_sc_mesh(),
        scratch_types=[
            pltpu.VMEM((2, _KB, 128), jnp.int32),
            pltpu.VMEM((2, _KB, 128), jnp.int32),
            pltpu.VMEM((2, 128, 32), jnp.float32),
            pltpu.VMEM_SHARED((_ACC, 32), jnp.float32),
        ] + [pltpu.SemaphoreType.DMA] * 9,
        compiler_params=pltpu.CompilerParams(use_tc_tiling_on_sc=False),
    )(h_flat, gidx, didx, z32)


def _bn_ab(s_ref, q_ref, g_ref, be_ref):
    mu = s_ref[...] / float(_N)
    var = q_ref[...] / float(_N) - mu * mu
    a = g_ref[...] * lax.rsqrt(var + 1e-5)
    b = be_ref[...] - mu * a
    return jnp.concatenate([a, b], axis=0)


def _mm1_body(x_ref, a0_ref, w_ref, g_ref, be_ref, t_ref, ab_ref, s_ref, q_ref):
    i = pl.program_id(0)

    @pl.when(i == 0)
    def _():
        s_ref[...] = jnp.zeros_like(s_ref)
        q_ref[...] = jnp.zeros_like(q_ref)

    xa = x_ref[...] + a0_ref[:, :16] + a0_ref[:, 16:]
    t = jnp.dot(xa, w_ref[...], preferred_element_type=jnp.float32)
    t_ref[...] = t
    s_ref[...] += jnp.sum(t, axis=0, keepdims=True)
    q_ref[...] += jnp.sum(t * t, axis=0, keepdims=True)

    @pl.when(i == _NB - 1)
    def _():
        ab_ref[...] = _bn_ab(s_ref, q_ref, g_ref, be_ref)


def _mm1(x_pad, a0, w1p, g, be):
    return pl.pallas_call(
        _mm1_body,
        grid=(_NB,),
        in_specs=[
            pl.BlockSpec((_BN, 16), lambda i: (i, 0)),
            pl.BlockSpec((_BN, 32), lambda i: (i, 0)),
            pl.BlockSpec((16, _H), lambda i: (0, 0)),
            pl.BlockSpec((1, _H), lambda i: (0, 0)),
            pl.BlockSpec((1, _H), lambda i: (0, 0)),
        ],
        out_specs=[
            pl.BlockSpec((_BN, _H), lambda i: (i, 0)),
            pl.BlockSpec((2, _H), lambda i: (0, 0)),
        ],
        out_shape=[
            jax.ShapeDtypeStruct((_N, _H), jnp.float32),
            jax.ShapeDtypeStruct((2, _H), jnp.float32),
        ],
        scratch_shapes=[
            pltpu.VMEM((1, _H), jnp.float32),
            pltpu.VMEM((1, _H), jnp.float32),
        ],
    )(x_pad, a0, w1p, g, be)


def _mm2_body(h_ref, ag_ref, w_ref, g_ref, be_ref, t_ref, ab_ref, s_ref, q_ref):
    i = pl.program_id(0)

    @pl.when(i == 0)
    def _():
        s_ref[...] = jnp.zeros_like(s_ref)
        q_ref[...] = jnp.zeros_like(q_ref)

    t = jnp.dot(h_ref[...] + ag_ref[...], w_ref[...],
                preferred_element_type=jnp.float32)
    t_ref[...] = t
    s_ref[...] += jnp.sum(t, axis=0, keepdims=True)
    q_ref[...] += jnp.sum(t * t, axis=0, keepdims=True)

    @pl.when(i == _NB - 1)
    def _():
        ab_ref[...] = _bn_ab(s_ref, q_ref, g_ref, be_ref)


def _mm2(h4, agg4, w, g, be):
    return pl.pallas_call(
        _mm2_body,
        grid=(_NB,),
        in_specs=[
            pl.BlockSpec((_BN, _H), lambda i: (i, 0)),
            pl.BlockSpec((_BN, _H), lambda i: (i, 0)),
            pl.BlockSpec((_H, _H), lambda i: (0, 0)),
            pl.BlockSpec((1, _H), lambda i: (0, 0)),
            pl.BlockSpec((1, _H), lambda i: (0, 0)),
        ],
        out_specs=[
            pl.BlockSpec((_BN, _H), lambda i: (i, 0)),
            pl.BlockSpec((2, _H), lambda i: (0, 0)),
        ],
        out_shape=[
            jax.ShapeDtypeStruct((_N, _H), jnp.float32),
            jax.ShapeDtypeStruct((2, _H), jnp.float32),
        ],
        scratch_shapes=[
            pltpu.VMEM((1, _H), jnp.float32),
            pltpu.VMEM((1, _H), jnp.float32),
        ],
    )(h4, agg4, w, g, be)


def _act_body(t_ref, ab_ref, o_ref):
    o_ref[...] = jnp.maximum(t_ref[...] * ab_ref[0:1, :] + ab_ref[1:2, :], 0.0)


def _act(t, ab):
    return pl.pallas_call(
        _act_body,
        grid=(_NB,),
        in_specs=[
            pl.BlockSpec((_BN, _H), lambda i: (i, 0)),
            pl.BlockSpec((2, _H), lambda i: (0, 0)),
        ],
        out_specs=pl.BlockSpec((_BN, _H), lambda i: (i, 0)),
        out_shape=jax.ShapeDtypeStruct((_N, _H), jnp.float32),
        compiler_params=pltpu.CompilerParams(
            dimension_semantics=("parallel",)),
    )(t, ab)


def _pool_body(t_ref, ab_ref, b_ref, wfc_ref, o_ref, s_ref, c_ref):
    i = pl.program_id(0)

    @pl.when(i == 0)
    def _():
        s_ref[...] = jnp.zeros_like(s_ref)
        c_ref[...] = jnp.zeros_like(c_ref)

    h = jnp.maximum(t_ref[...] * ab_ref[0:1, :] + ab_ref[1:2, :], 0.0)
    ohT = (lax.broadcasted_iota(jnp.int32, (_G, _BN), 0) == b_ref[0]
           ).astype(jnp.float32)
    s_ref[...] += jnp.dot(ohT, h, preferred_element_type=jnp.float32)
    c_ref[...] += jnp.dot(ohT, jnp.ones_like(h),
                          preferred_element_type=jnp.float32)

    @pl.when(i == _NB - 1)
    def _():
        pooled = s_ref[...] / jnp.maximum(c_ref[...], 1.0)
        z = jnp.dot(pooled, wfc_ref[...], preferred_element_type=jnp.float32)
        o_ref[...] = 1.0 / (1.0 + jnp.exp(-z))


def _pool(t3, ab3, batch_r, wfc_p):
    return pl.pallas_call(
        _pool_body,
        grid=(_NB,),
        in_specs=[
            pl.BlockSpec((_BN, _H), lambda i: (i, 0)),
            pl.BlockSpec((2, _H), lambda i: (0, 0)),
            pl.BlockSpec((1, 1, _BN), lambda i: (i, 0, 0)),
            pl.BlockSpec((_H, _H), lambda i: (0, 0)),
        ],
        out_specs=pl.BlockSpec((_G, _H), lambda i: (0, 0)),
        out_shape=jax.ShapeDtypeStruct((_G, _H), jnp.float32),
        scratch_shapes=[
            pltpu.VMEM((_G, _H), jnp.float32),
            pltpu.VMEM((_G, _H), jnp.float32),
        ],
    )(t3, ab3, batch_r, wfc_p)


def kernel(x, edge_index, batch, W1, g1, b1, W2, g2, b2, W3, g3, b3, Wfc):
    src = edge_index[0]
    dst = edge_index[1]
    pad = _EP - _E
    src_p = jnp.concatenate([src, jnp.zeros((pad,), jnp.int32)])
    dst_p = jnp.concatenate([dst, jnp.full((pad,), _N, jnp.int32)])
    pad1 = _EP1 - _E
    sidx = jnp.concatenate([src, jnp.zeros((pad1,), jnp.int32)]).reshape(_R1, 128)
    didx1 = jnp.concatenate([dst, jnp.full((pad1,), _N, jnp.int32)]).reshape(_R1, 128)
    didx = dst_p.reshape(_R, 128)
    gidx = (src_p[None, :] * 4
            + jnp.arange(4, dtype=jnp.int32)[:, None]).reshape(4 * _R, 128)
    z16 = jnp.zeros((_APT, 16), jnp.float32)
    z32 = jnp.zeros((_APT, 32), jnp.float32)

    x_pad = jnp.pad(x, ((0, 0), (0, 12)))
    w1p = jnp.pad(W1, ((0, 12), (0, 0)))
    wfc_p = jnp.pad(Wfc, ((0, 0), (0, _H - 1)))

    a0 = _agg16(x_pad, sidx, didx1, z16)
    t1, ab1 = _mm1(x_pad, a0, w1p, g1[None], b1[None])
    h1 = _act(t1, ab1)
    agg1 = _agg32(h1.reshape(4 * _N, 32), gidx, didx, z32)
    t2, ab2 = _mm2(h1, agg1, W2, g2[None], b2[None])
    h2 = _act(t2, ab2)
    agg2 = _agg32(h2.reshape(4 * _N, 32), gidx, didx, z32)
    t3, ab3 = _mm2(h2, agg2, W3, g3[None], b3[None])
    out = _pool(t3, ab3, batch.reshape(_NB, 1, _BN), wfc_p)
    return out[:, :1]


# speedup vs baseline: 2.0325x; 1.3139x over previous
import jax
import jax.numpy as jnp
from jax import lax
from jax.experimental import pallas as pl
from jax.experimental.pallas import tpu as pltpu
from jax.experimental.pallas import tpu_sc as plsc

_N = 50000
_E = 800000
_H = 128
_G = 64
_R = 6272
_EP = _R * 128
_RPT = _R // 16
_R1 = 6400
_EP1 = _R1 * 128
_RPT1 = _R1 // 32
_ACC = 51200
_APT = _ACC // 16
_KB = 8
_BN = 1000
_NB = _N // _BN

def _sc_mesh():
    return plsc.VectorSubcoreMesh(
        core_axis_name="c", subcore_axis_name="s", num_cores=2, num_subcores=16)


def _edge_pipeline(tbl_hbm, gidx_hbm, didx_hbm, gbase, dbase, ngrp,
                   gidx_v, didx_v, rows_v, acc, sg, ss, semi):
    pltpu.sync_copy(gidx_hbm.at[pl.ds(gbase, _KB)], gidx_v.at[0])
    pltpu.sync_copy(didx_hbm.at[pl.ds(dbase, _KB)], didx_v.at[0])
    plsc.subcore_barrier()

    @pl.loop(0, ngrp)
    def _grp(g):
        b = lax.rem(g, 2)
        nb = 1 - b
        gn = lax.rem(g + 1, ngrp)
        gi = pltpu.async_copy(
            gidx_hbm.at[pl.ds(gbase + gn * _KB, _KB)], gidx_v.at[nb], semi)
        di = pltpu.async_copy(
            didx_hbm.at[pl.ds(dbase + gn * _KB, _KB)], didx_v.at[nb], semi)
        dg = {}
        ds = {}
        for j in range(2):
            dg[j] = pltpu.async_copy(
                tbl_hbm.at[gidx_v.at[b, j]], rows_v.at[j % 4], sg[j % 4])
        for j in range(_KB):
            dg[j].wait()
            ds[j] = pltpu.async_copy(
                rows_v.at[j % 4], acc.at[didx_v.at[b, j]], ss[j % 4],
                add=True)
            if j + 2 < _KB:
                if j - 2 >= 0:
                    ds[j - 2].wait()
                dg[j + 2] = pltpu.async_copy(
                    tbl_hbm.at[gidx_v.at[b, j + 2]],
                    rows_v.at[(j + 2) % 4], sg[(j + 2) % 4])
        for j in range(max(0, _KB - 4), _KB):
            ds[j].wait()
        gi.wait()
        di.wait()

    plsc.subcore_barrier()


def _agg16_body(x_hbm, sidx_hbm, didx_hbm, z_hbm, out_hbm,
                sidx_v, didx_v, rows_v, acc,
                sg0, sg1, sg2, sg3, ss0, ss1, ss2, ss3, semi):
    c = lax.axis_index("c")
    s = lax.axis_index("s")
    pltpu.sync_copy(z_hbm, acc.at[pl.ds(s * _APT, _APT)])
    base = c * (16 * _RPT1) + s * _RPT1
    _edge_pipeline(x_hbm, sidx_hbm, didx_hbm, base, base, _RPT1 // _KB,
                   sidx_v, didx_v, rows_v, acc,
                   (sg0, sg1, sg2, sg3), (ss0, ss1, ss2, ss3), semi)
    pltpu.sync_copy(acc.at[pl.ds(s * _APT, _APT)],
                    out_hbm.at[pl.ds(s * _APT, _APT), pl.ds(c * 16, 16)])


def _agg16(x_pad, sidx, didx, z16):
    return pl.kernel(
        _agg16_body,
        out_type=jax.ShapeDtypeStruct((_ACC, 32), jnp.float32),
        mesh=_sc_mesh(),
        scratch_types=[
            pltpu.VMEM((2, _KB, 128), jnp.int32),
            pltpu.VMEM((2, _KB, 128), jnp.int32),
            pltpu.VMEM((4, 128, 16), jnp.float32),
            pltpu.VMEM_SHARED((_ACC, 16), jnp.float32),
        ] + [pltpu.SemaphoreType.DMA] * 9,
        compiler_params=pltpu.CompilerParams(use_tc_tiling_on_sc=False),
    )(x_pad, sidx, didx, z16)


def _agg32_body(h_hbm, gidx_hbm, didx_hbm, z_hbm, out_hbm,
                gidx_v, didx_v, rows_v, acc,
                sg0, sg1, sg2, sg3, ss0, ss1, ss2, ss3, semi):
    c = lax.axis_index("c")
    s = lax.axis_index("s")
    for k in range(2):
        cc = c * 2 + k
        pltpu.sync_copy(z_hbm, acc.at[pl.ds(s * _APT, _APT)])
        _edge_pipeline(h_hbm, gidx_hbm, didx_hbm,
                       cc * _R + s * _RPT, s * _RPT, _RPT // _KB,
                       gidx_v, didx_v, rows_v, acc,
                       (sg0, sg1, sg2, sg3), (ss0, ss1, ss2, ss3), semi)
        pltpu.sync_copy(acc.at[pl.ds(s * _APT, _APT)],
                        out_hbm.at[pl.ds(s * _APT, _APT), pl.ds(cc * 32, 32)])


def _agg32(h_flat, gidx, didx, z32):
    return pl.kernel(
        _agg32_body,
        out_type=jax.ShapeDtypeStruct((_ACC, _H), jnp.float32),
        mesh=_sc_mesh(),
        scratch_types=[
            pltpu.VMEM((2, _KB, 128), jnp.int32),
            pltpu.VMEM((2, _KB, 128), jnp.int32),
            pltpu.VMEM((4, 128, 32), jnp.float32),
            pltpu.VMEM_SHARED((_ACC, 32), jnp.float32),
        ] + [pltpu.SemaphoreType.DMA] * 9,
        compiler_params=pltpu.CompilerParams(use_tc_tiling_on_sc=False),
    )(h_flat, gidx, didx, z32)


def _bn_ab(s_ref, q_ref, g_ref, be_ref):
    mu = s_ref[...] / float(_N)
    var = q_ref[...] / float(_N) - mu * mu
    a = g_ref[...] * lax.rsqrt(var + 1e-5)
    b = be_ref[...] - mu * a
    return jnp.concatenate([a, b], axis=0)


def _mm1_body(x_ref, a0_ref, w_ref, g_ref, be_ref, t_ref, ab_ref, s_ref, q_ref):
    i = pl.program_id(0)

    @pl.when(i == 0)
    def _():
        s_ref[...] = jnp.zeros_like(s_ref)
        q_ref[...] = jnp.zeros_like(q_ref)

    xa = x_ref[...] + a0_ref[:, :16] + a0_ref[:, 16:]
    t = jnp.dot(xa, w_ref[...], preferred_element_type=jnp.float32)
    t_ref[...] = t
    s_ref[...] += jnp.sum(t, axis=0, keepdims=True)
    q_ref[...] += jnp.sum(t * t, axis=0, keepdims=True)

    @pl.when(i == _NB - 1)
    def _():
        ab_ref[...] = _bn_ab(s_ref, q_ref, g_ref, be_ref)


def _mm1(x_pad, a0, w1p, g, be):
    return pl.pallas_call(
        _mm1_body,
        grid=(_NB,),
        in_specs=[
            pl.BlockSpec((_BN, 16), lambda i: (i, 0)),
            pl.BlockSpec((_BN, 32), lambda i: (i, 0)),
            pl.BlockSpec((16, _H), lambda i: (0, 0)),
            pl.BlockSpec((1, _H), lambda i: (0, 0)),
            pl.BlockSpec((1, _H), lambda i: (0, 0)),
        ],
        out_specs=[
            pl.BlockSpec((_BN, _H), lambda i: (i, 0)),
            pl.BlockSpec((2, _H), lambda i: (0, 0)),
        ],
        out_shape=[
            jax.ShapeDtypeStruct((_N, _H), jnp.float32),
            jax.ShapeDtypeStruct((2, _H), jnp.float32),
        ],
        scratch_shapes=[
            pltpu.VMEM((1, _H), jnp.float32),
            pltpu.VMEM((1, _H), jnp.float32),
        ],
    )(x_pad, a0, w1p, g, be)


def _mm2_body(h_ref, ag_ref, w_ref, g_ref, be_ref, t_ref, ab_ref, s_ref, q_ref):
    i = pl.program_id(0)

    @pl.when(i == 0)
    def _():
        s_ref[...] = jnp.zeros_like(s_ref)
        q_ref[...] = jnp.zeros_like(q_ref)

    t = jnp.dot(h_ref[...] + ag_ref[...], w_ref[...],
                preferred_element_type=jnp.float32)
    t_ref[...] = t
    s_ref[...] += jnp.sum(t, axis=0, keepdims=True)
    q_ref[...] += jnp.sum(t * t, axis=0, keepdims=True)

    @pl.when(i == _NB - 1)
    def _():
        ab_ref[...] = _bn_ab(s_ref, q_ref, g_ref, be_ref)


def _mm2(h4, agg4, w, g, be):
    return pl.pallas_call(
        _mm2_body,
        grid=(_NB,),
        in_specs=[
            pl.BlockSpec((_BN, _H), lambda i: (i, 0)),
            pl.BlockSpec((_BN, _H), lambda i: (i, 0)),
            pl.BlockSpec((_H, _H), lambda i: (0, 0)),
            pl.BlockSpec((1, _H), lambda i: (0, 0)),
            pl.BlockSpec((1, _H), lambda i: (0, 0)),
        ],
        out_specs=[
            pl.BlockSpec((_BN, _H), lambda i: (i, 0)),
            pl.BlockSpec((2, _H), lambda i: (0, 0)),
        ],
        out_shape=[
            jax.ShapeDtypeStruct((_N, _H), jnp.float32),
            jax.ShapeDtypeStruct((2, _H), jnp.float32),
        ],
        scratch_shapes=[
            pltpu.VMEM((1, _H), jnp.float32),
            pltpu.VMEM((1, _H), jnp.float32),
        ],
    )(h4, agg4, w, g, be)


def _act_body(t_ref, ab_ref, o_ref):
    o_ref[...] = jnp.maximum(t_ref[...] * ab_ref[0:1, :] + ab_ref[1:2, :], 0.0)


def _act(t, ab):
    return pl.pallas_call(
        _act_body,
        grid=(_NB,),
        in_specs=[
            pl.BlockSpec((_BN, _H), lambda i: (i, 0)),
            pl.BlockSpec((2, _H), lambda i: (0, 0)),
        ],
        out_specs=pl.BlockSpec((_BN, _H), lambda i: (i, 0)),
        out_shape=jax.ShapeDtypeStruct((_N, _H), jnp.float32),
        compiler_params=pltpu.CompilerParams(
            dimension_semantics=("parallel",)),
    )(t, ab)


def _pool_body(t_ref, ab_ref, b_ref, wfc_ref, o_ref, s_ref, c_ref):
    i = pl.program_id(0)

    @pl.when(i == 0)
    def _():
        s_ref[...] = jnp.zeros_like(s_ref)
        c_ref[...] = jnp.zeros_like(c_ref)

    h = jnp.maximum(t_ref[...] * ab_ref[0:1, :] + ab_ref[1:2, :], 0.0)
    ohT = (lax.broadcasted_iota(jnp.int32, (_G, _BN), 0) == b_ref[0]
           ).astype(jnp.float32)
    s_ref[...] += jnp.dot(ohT, h, preferred_element_type=jnp.float32)
    c_ref[...] += jnp.dot(ohT, jnp.ones_like(h),
                          preferred_element_type=jnp.float32)

    @pl.when(i == _NB - 1)
    def _():
        pooled = s_ref[...] / jnp.maximum(c_ref[...], 1.0)
        z = jnp.dot(pooled, wfc_ref[...], preferred_element_type=jnp.float32)
        o_ref[...] = 1.0 / (1.0 + jnp.exp(-z))


def _pool(t3, ab3, batch_r, wfc_p):
    return pl.pallas_call(
        _pool_body,
        grid=(_NB,),
        in_specs=[
            pl.BlockSpec((_BN, _H), lambda i: (i, 0)),
            pl.BlockSpec((2, _H), lambda i: (0, 0)),
            pl.BlockSpec((1, 1, _BN), lambda i: (i, 0, 0)),
            pl.BlockSpec((_H, _H), lambda i: (0, 0)),
        ],
        out_specs=pl.BlockSpec((_G, _H), lambda i: (0, 0)),
        out_shape=jax.ShapeDtypeStruct((_G, _H), jnp.float32),
        scratch_shapes=[
            pltpu.VMEM((_G, _H), jnp.float32),
            pltpu.VMEM((_G, _H), jnp.float32),
        ],
    )(t3, ab3, batch_r, wfc_p)


def kernel(x, edge_index, batch, W1, g1, b1, W2, g2, b2, W3, g3, b3, Wfc):
    src = edge_index[0]
    dst = edge_index[1]
    pad = _EP - _E
    src_p = jnp.concatenate([src, jnp.zeros((pad,), jnp.int32)])
    dst_p = jnp.concatenate([dst, jnp.full((pad,), _N, jnp.int32)])
    pad1 = _EP1 - _E
    sidx = jnp.concatenate([src, jnp.zeros((pad1,), jnp.int32)]).reshape(_R1, 128)
    didx1 = jnp.concatenate([dst, jnp.full((pad1,), _N, jnp.int32)]).reshape(_R1, 128)
    didx = dst_p.reshape(_R, 128)
    gidx = (src_p[None, :] * 4
            + jnp.arange(4, dtype=jnp.int32)[:, None]).reshape(4 * _R, 128)
    z16 = jnp.zeros((_APT, 16), jnp.float32)
    z32 = jnp.zeros((_APT, 32), jnp.float32)

    x_pad = jnp.pad(x, ((0, 0), (0, 12)))
    w1p = jnp.pad(W1, ((0, 12), (0, 0)))
    wfc_p = jnp.pad(Wfc, ((0, 0), (0, _H - 1)))

    a0 = _agg16(x_pad, sidx, didx1, z16)
    t1, ab1 = _mm1(x_pad, a0, w1p, g1[None], b1[None])
    h1 = _act(t1, ab1)
    agg1 = _agg32(h1.reshape(4 * _N, 32), gidx, didx, z32)
    t2, ab2 = _mm2(h1, agg1, W2, g2[None], b2[None])
    h2 = _act(t2, ab2)
    agg2 = _agg32(h2.reshape(4 * _N, 32), gidx, didx, z32)
    t3, ab3 = _mm2(h2, agg2, W3, g3[None], b3[None])
    out = _pool(t3, ab3, batch.reshape(_NB, 1, _BN), wfc_p)
    return out[:, :1]


# speedup vs baseline: 2.1774x; 1.0713x over previous
import jax
import jax.numpy as jnp
from jax import lax
from jax.experimental import pallas as pl
from jax.experimental.pallas import tpu as pltpu
from jax.experimental.pallas import tpu_sc as plsc

_N = 50000
_E = 800000
_H = 128
_G = 64
_R = 6272
_EP = _R * 128
_RPT = _R // 16
_R1 = 6400
_EP1 = _R1 * 128
_RPT1 = _R1 // 32
_ACC = 50048
_APT = _ACC // 16
_KB = 8
_RB = 6
_BN = 1000
_NB = _N // _BN

def _sc_mesh():
    return plsc.VectorSubcoreMesh(
        core_axis_name="c", subcore_axis_name="s", num_cores=2, num_subcores=16)


def _edge_pipeline(tbl_hbm, gidx_hbm, didx_hbm, gbase, dbase, ngrp,
                   gidx_v, didx_v, rows_v, acc, sg, ss, semi):
    pltpu.sync_copy(gidx_hbm.at[pl.ds(gbase, _KB)], gidx_v.at[0])
    pltpu.sync_copy(didx_hbm.at[pl.ds(dbase, _KB)], didx_v.at[0])
    plsc.subcore_barrier()

    @pl.loop(0, ngrp)
    def _grp(g):
        b = lax.rem(g, 2)
        nb = 1 - b
        gn = lax.rem(g + 1, ngrp)
        gi = pltpu.async_copy(
            gidx_hbm.at[pl.ds(gbase + gn * _KB, _KB)], gidx_v.at[nb], semi)
        di = pltpu.async_copy(
            didx_hbm.at[pl.ds(dbase + gn * _KB, _KB)], didx_v.at[nb], semi)
        dg = {}
        ds = {}
        for j in range(3):
            dg[j] = pltpu.async_copy(
                tbl_hbm.at[gidx_v.at[b, j]], rows_v.at[j % _RB], sg[j % _RB])
        for j in range(_KB):
            dg[j].wait()
            ds[j] = pltpu.async_copy(
                rows_v.at[j % _RB], acc.at[didx_v.at[b, j]], ss[j % _RB],
                add=True)
            if j + 3 < _KB:
                if j - 3 >= 0:
                    ds[j - 3].wait()
                dg[j + 3] = pltpu.async_copy(
                    tbl_hbm.at[gidx_v.at[b, j + 3]],
                    rows_v.at[(j + 3) % _RB], sg[(j + 3) % _RB])
        for j in range(max(0, _KB - _RB), _KB):
            ds[j].wait()
        gi.wait()
        di.wait()

    plsc.subcore_barrier()


def _agg16_body(x_hbm, sidx_hbm, didx_hbm, z_hbm, out_hbm,
                sidx_v, didx_v, rows_v, acc,
                sg0, sg1, sg2, sg3, sg4, sg5,
                ss0, ss1, ss2, ss3, ss4, ss5, semi):
    c = lax.axis_index("c")
    s = lax.axis_index("s")
    pltpu.sync_copy(z_hbm, acc.at[pl.ds(s * _APT, _APT)])
    base = c * (16 * _RPT1) + s * _RPT1
    _edge_pipeline(x_hbm, sidx_hbm, didx_hbm, base, base, _RPT1 // _KB,
                   sidx_v, didx_v, rows_v, acc,
                   (sg0, sg1, sg2, sg3, sg4, sg5),
                   (ss0, ss1, ss2, ss3, ss4, ss5), semi)
    pltpu.sync_copy(acc.at[pl.ds(s * _APT, _APT)],
                    out_hbm.at[pl.ds(s * _APT, _APT), pl.ds(c * 16, 16)])


def _agg16(x_pad, sidx, didx, z16):
    return pl.kernel(
        _agg16_body,
        out_type=jax.ShapeDtypeStruct((_ACC, 32), jnp.float32),
        mesh=_sc_mesh(),
        scratch_types=[
            pltpu.VMEM((2, _KB, 128), jnp.int32),
            pltpu.VMEM((2, _KB, 128), jnp.int32),
            pltpu.VMEM((_RB, 128, 16), jnp.float32),
            pltpu.VMEM_SHARED((_ACC, 16), jnp.float32),
        ] + [pltpu.SemaphoreType.DMA] * 13,
        compiler_params=pltpu.CompilerParams(use_tc_tiling_on_sc=False),
    )(x_pad, sidx, didx, z16)


def _agg32_body(h_hbm, gidx_hbm, didx_hbm, z_hbm, out_hbm,
                gidx_v, didx_v, rows_v, acc,
                sg0, sg1, sg2, sg3, sg4, sg5,
                ss0, ss1, ss2, ss3, ss4, ss5, semi):
    c = lax.axis_index("c")
    s = lax.axis_index("s")
    for k in range(2):
        cc = c * 2 + k
        pltpu.sync_copy(z_hbm, acc.at[pl.ds(s * _APT, _APT)])
        _edge_pipeline(h_hbm, gidx_hbm, didx_hbm,
                       cc * _R + s * _RPT, s * _RPT, _RPT // _KB,
                       gidx_v, didx_v, rows_v, acc,
                       (sg0, sg1, sg2, sg3, sg4, sg5),
                       (ss0, ss1, ss2, ss3, ss4, ss5), semi)
        pltpu.sync_copy(acc.at[pl.ds(s * _APT, _APT)],
                        out_hbm.at[pl.ds(s * _APT, _APT), pl.ds(cc * 32, 32)])


def _agg32(h_flat, gidx, didx, z32):
    return pl.kernel(
        _agg32_body,
        out_type=jax.ShapeDtypeStruct((_ACC, _H), jnp.float32),
        mesh=_sc_mesh(),
        scratch_types=[
            pltpu.VMEM((2, _KB, 128), jnp.int32),
            pltpu.VMEM((2, _KB, 128), jnp.int32),
            pltpu.VMEM((_RB, 128, 32), jnp.float32),
            pltpu.VMEM_SHARED((_ACC, 32), jnp.float32),
        ] + [pltpu.SemaphoreType.DMA] * 13,
        compiler_params=pltpu.CompilerParams(use_tc_tiling_on_sc=False),
    )(h_flat, gidx, didx, z32)


def _bn_ab(s_ref, q_ref, g_ref, be_ref):
    mu = s_ref[...] / float(_N)
    var = q_ref[...] / float(_N) - mu * mu
    a = g_ref[...] * lax.rsqrt(var + 1e-5)
    b = be_ref[...] - mu * a
    return jnp.concatenate([a, b], axis=0)


def _mm1_body(x_ref, a0_ref, w_ref, g_ref, be_ref, t_ref, ab_ref, s_ref, q_ref):
    i = pl.program_id(0)

    @pl.when(i == 0)
    def _():
        s_ref[...] = jnp.zeros_like(s_ref)
        q_ref[...] = jnp.zeros_like(q_ref)

    xa = x_ref[...] + a0_ref[:, :16] + a0_ref[:, 16:]
    t = jnp.dot(xa, w_ref[...], preferred_element_type=jnp.float32)
    t_ref[...] = t
    s_ref[...] += jnp.sum(t, axis=0, keepdims=True)
    q_ref[...] += jnp.sum(t * t, axis=0, keepdims=True)

    @pl.when(i == _NB - 1)
    def _():
        ab_ref[...] = _bn_ab(s_ref, q_ref, g_ref, be_ref)


def _mm1(x_pad, a0, w1p, g, be):
    return pl.pallas_call(
        _mm1_body,
        grid=(_NB,),
        in_specs=[
            pl.BlockSpec((_BN, 16), lambda i: (i, 0)),
            pl.BlockSpec((_BN, 32), lambda i: (i, 0)),
            pl.BlockSpec((16, _H), lambda i: (0, 0)),
            pl.BlockSpec((1, _H), lambda i: (0, 0)),
            pl.BlockSpec((1, _H), lambda i: (0, 0)),
        ],
        out_specs=[
            pl.BlockSpec((_BN, _H), lambda i: (i, 0)),
            pl.BlockSpec((2, _H), lambda i: (0, 0)),
        ],
        out_shape=[
            jax.ShapeDtypeStruct((_N, _H), jnp.float32),
            jax.ShapeDtypeStruct((2, _H), jnp.float32),
        ],
        scratch_shapes=[
            pltpu.VMEM((1, _H), jnp.float32),
            pltpu.VMEM((1, _H), jnp.float32),
        ],
    )(x_pad, a0, w1p, g, be)


def _mm2_body(h_ref, ag_ref, w_ref, g_ref, be_ref, t_ref, ab_ref, s_ref, q_ref):
    i = pl.program_id(0)

    @pl.when(i == 0)
    def _():
        s_ref[...] = jnp.zeros_like(s_ref)
        q_ref[...] = jnp.zeros_like(q_ref)

    t = jnp.dot(h_ref[...] + ag_ref[...], w_ref[...],
                preferred_element_type=jnp.float32)
    t_ref[...] = t
    s_ref[...] += jnp.sum(t, axis=0, keepdims=True)
    q_ref[...] += jnp.sum(t * t, axis=0, keepdims=True)

    @pl.when(i == _NB - 1)
    def _():
        ab_ref[...] = _bn_ab(s_ref, q_ref, g_ref, be_ref)


def _mm2(h4, agg4, w, g, be):
    return pl.pallas_call(
        _mm2_body,
        grid=(_NB,),
        in_specs=[
            pl.BlockSpec((_BN, _H), lambda i: (i, 0)),
            pl.BlockSpec((_BN, _H), lambda i: (i, 0)),
            pl.BlockSpec((_H, _H), lambda i: (0, 0)),
            pl.BlockSpec((1, _H), lambda i: (0, 0)),
            pl.BlockSpec((1, _H), lambda i: (0, 0)),
        ],
        out_specs=[
            pl.BlockSpec((_BN, _H), lambda i: (i, 0)),
            pl.BlockSpec((2, _H), lambda i: (0, 0)),
        ],
        out_shape=[
            jax.ShapeDtypeStruct((_N, _H), jnp.float32),
            jax.ShapeDtypeStruct((2, _H), jnp.float32),
        ],
        scratch_shapes=[
            pltpu.VMEM((1, _H), jnp.float32),
            pltpu.VMEM((1, _H), jnp.float32),
        ],
    )(h4, agg4, w, g, be)


def _act_body(t_ref, ab_ref, o_ref):
    o_ref[...] = jnp.maximum(t_ref[...] * ab_ref[0:1, :] + ab_ref[1:2, :], 0.0)


def _act(t, ab):
    return pl.pallas_call(
        _act_body,
        grid=(_NB,),
        in_specs=[
            pl.BlockSpec((_BN, _H), lambda i: (i, 0)),
            pl.BlockSpec((2, _H), lambda i: (0, 0)),
        ],
        out_specs=pl.BlockSpec((_BN, _H), lambda i: (i, 0)),
        out_shape=jax.ShapeDtypeStruct((_N, _H), jnp.float32),
        compiler_params=pltpu.CompilerParams(
            dimension_semantics=("parallel",)),
    )(t, ab)


def _pool_body(t_ref, ab_ref, b_ref, wfc_ref, o_ref, s_ref, c_ref):
    i = pl.program_id(0)

    @pl.when(i == 0)
    def _():
        s_ref[...] = jnp.zeros_like(s_ref)
        c_ref[...] = jnp.zeros_like(c_ref)

    h = jnp.maximum(t_ref[...] * ab_ref[0:1, :] + ab_ref[1:2, :], 0.0)
    ohT = (lax.broadcasted_iota(jnp.int32, (_G, _BN), 0) == b_ref[0]
           ).astype(jnp.float32)
    s_ref[...] += jnp.dot(ohT, h, preferred_element_type=jnp.float32)
    c_ref[...] += jnp.dot(ohT, jnp.ones_like(h),
                          preferred_element_type=jnp.float32)

    @pl.when(i == _NB - 1)
    def _():
        pooled = s_ref[...] / jnp.maximum(c_ref[...], 1.0)
        z = jnp.dot(pooled, wfc_ref[...], preferred_element_type=jnp.float32)
        o_ref[...] = 1.0 / (1.0 + jnp.exp(-z))


def _pool(t3, ab3, batch_r, wfc_p):
    return pl.pallas_call(
        _pool_body,
        grid=(_NB,),
        in_specs=[
            pl.BlockSpec((_BN, _H), lambda i: (i, 0)),
            pl.BlockSpec((2, _H), lambda i: (0, 0)),
            pl.BlockSpec((1, 1, _BN), lambda i: (i, 0, 0)),
            pl.BlockSpec((_H, _H), lambda i: (0, 0)),
        ],
        out_specs=pl.BlockSpec((_G, _H), lambda i: (0, 0)),
        out_shape=jax.ShapeDtypeStruct((_G, _H), jnp.float32),
        scratch_shapes=[
            pltpu.VMEM((_G, _H), jnp.float32),
            pltpu.VMEM((_G, _H), jnp.float32),
        ],
    )(t3, ab3, batch_r, wfc_p)


def kernel(x, edge_index, batch, W1, g1, b1, W2, g2, b2, W3, g3, b3, Wfc):
    src = edge_index[0]
    dst = edge_index[1]
    pad = _EP - _E
    src_p = jnp.concatenate([src, jnp.zeros((pad,), jnp.int32)])
    dst_p = jnp.concatenate([dst, jnp.full((pad,), _N, jnp.int32)])
    pad1 = _EP1 - _E
    sidx = jnp.concatenate([src, jnp.zeros((pad1,), jnp.int32)]).reshape(_R1, 128)
    didx1 = jnp.concatenate([dst, jnp.full((pad1,), _N, jnp.int32)]).reshape(_R1, 128)
    didx = dst_p.reshape(_R, 128)
    gidx = (src_p[None, :] * 4
            + jnp.arange(4, dtype=jnp.int32)[:, None]).reshape(4 * _R, 128)
    z16 = jnp.zeros((_APT, 16), jnp.float32)
    z32 = jnp.zeros((_APT, 32), jnp.float32)

    x_pad = jnp.pad(x, ((0, 0), (0, 12)))
    w1p = jnp.pad(W1, ((0, 12), (0, 0)))
    wfc_p = jnp.pad(Wfc, ((0, 0), (0, _H - 1)))

    a0 = _agg16(x_pad, sidx, didx1, z16)
    t1, ab1 = _mm1(x_pad, a0, w1p, g1[None], b1[None])
    h1 = _act(t1, ab1)
    agg1 = _agg32(h1.reshape(4 * _N, 32), gidx, didx, z32)
    t2, ab2 = _mm2(h1, agg1, W2, g2[None], b2[None])
    h2 = _act(t2, ab2)
    agg2 = _agg32(h2.reshape(4 * _N, 32), gidx, didx, z32)
    t3, ab3 = _mm2(h2, agg2, W3, g3[None], b3[None])
    out = _pool(t3, ab3, batch.reshape(_NB, 1, _BN), wfc_p)
    return out[:, :1]
